# baseline (speedup 1.0000x reference)
import functools

import jax
import jax.numpy as jnp
from jax import lax
from jax.experimental import pallas as pl
from jax.experimental.pallas import tpu as pltpu
from jax.experimental.pallas import tpu_sc as plsc

N = 100000
D = 128
G = 4096
H_OUT = 12

NC = 2
NS = 16
NW = NC * NS
CHUNK = 128
NFULL = N // CHUNK
TAIL = N - NFULL * CHUNK
JMAX = 25
ROWS_PER_SID = G // NS


def _sc_segment_sum(node_feats, idx_t, zrow):
    mesh = plsc.VectorSubcoreMesh(
        core_axis_name="c", subcore_axis_name="s", num_cores=NC, num_subcores=NS
    )

    @functools.partial(
        pl.kernel,
        out_type=jax.ShapeDtypeStruct((NC, G, D), jnp.float32),
        mesh=mesh,
        scratch_types=[
            pltpu.VMEM((JMAX, CHUNK), jnp.int32),
            pltpu.VMEM((4, CHUNK, D), jnp.float32),
            pltpu.VMEM((CHUNK, D), jnp.float32),
            pltpu.VMEM_SHARED((G, D), jnp.float32),
            pltpu.SemaphoreType.DMA((4,)),
            pltpu.SemaphoreType.DMA,
        ],
    )
    def seg_sum(node_hbm, idx_hbm, zrow_hbm, out_hbm, ids_buf, rbuf, zbuf, acc,
                sem, sem_s):
        cid = lax.axis_index("c")
        sid = lax.axis_index("s")
        wid = cid * NS + sid

        for p in range(2):
            pltpu.async_copy(
                node_hbm.at[pl.ds((wid + NW * p) * CHUNK, CHUNK)],
                rbuf.at[p], sem.at[p],
            )
        pltpu.sync_copy(idx_hbm.at[wid], ids_buf)

        @pl.when(wid == 13)
        def _():
            pltpu.sync_copy(zrow_hbm, zbuf)

        base = sid * ROWS_PER_SID
        pltpu.sync_copy(zrow_hbm, acc.at[pl.ds(base, CHUNK)])
        pltpu.sync_copy(zrow_hbm, acc.at[pl.ds(base + CHUNK, CHUNK)])
        plsc.subcore_barrier()

        nfull = jnp.where(wid <= 12, JMAX, JMAX - 1)

        def body(j, carry):
            b = lax.rem(j, 4)

            @pl.when(j >= 2)
            def _():
                pltpu.make_async_copy(
                    rbuf.at[0], acc.at[ids_buf.at[0]], sem_s
                ).wait()

            @pl.when(j + 2 < nfull)
            def _():
                c2 = wid + NW * (j + 2)
                b2 = lax.rem(j + 2, 4)
                pltpu.async_copy(
                    node_hbm.at[pl.ds(c2 * CHUNK, CHUNK)], rbuf.at[b2], sem.at[b2]
                )

            pltpu.make_async_copy(
                node_hbm.at[pl.ds(0, CHUNK)], rbuf.at[b], sem.at[b]
            ).wait()
            pltpu.async_copy(rbuf.at[b], acc.at[ids_buf.at[j]], sem_s, add=True)
            return carry

        lax.fori_loop(0, nfull, body, 0)

        for _ in range(2):
            pltpu.make_async_copy(rbuf.at[0], acc.at[ids_buf.at[0]], sem_s).wait()

        @pl.when(wid == 13)
        def _():
            pltpu.sync_copy(
                node_hbm.at[pl.ds(NFULL * CHUNK, TAIL)], zbuf.at[pl.ds(0, TAIL)]
            )
            pltpu.sync_copy(zbuf, acc.at[ids_buf.at[JMAX - 1]], add=True)

        plsc.subcore_barrier()

        pltpu.sync_copy(
            acc.at[pl.ds(base, ROWS_PER_SID)],
            out_hbm.at[cid, pl.ds(base, ROWS_PER_SID)],
        )

    return seg_sum(node_feats, idx_t, zrow)


def _tc_body(p_ref, g_ref, b_ref, w1_ref, b1_ref, w2_ref, b2_ref, o_ref):
    x = p_ref[0] + p_ref[1]
    mean = jnp.mean(x, axis=0, keepdims=True)
    xc = x - mean
    var = jnp.mean(xc * xc, axis=0, keepdims=True)
    gn = xc * lax.rsqrt(var + 1e-5) * g_ref[...] + b_ref[...]
    h = jnp.dot(gn, w1_ref[...], preferred_element_type=jnp.float32) + b1_ref[...]
    h = jnp.maximum(h, 0.0)
    o_ref[...] = jnp.dot(h, w2_ref[...], preferred_element_type=jnp.float32) + b2_ref[...]


def _tc_bn_mlp(partials, gamma, beta, W1, b1, W2p, b2p):
    return pl.pallas_call(
        _tc_body,
        out_shape=jax.ShapeDtypeStruct((G, D), jnp.float32),
    )(partials, gamma, beta, W1, b1, W2p, b2p)


def kernel(node_feats, segment_ids, gamma, beta, W1, b1, W2, b2):
    ids32 = segment_ids.astype(jnp.int32)
    ids_pad = jnp.zeros((NW * JMAX * CHUNK,), jnp.int32).at[:N].set(ids32)
    idx_t = ids_pad.reshape(JMAX, NW, CHUNK).transpose(1, 0, 2)
    zrow = jnp.zeros((CHUNK, D), jnp.float32)

    partials = _sc_segment_sum(node_feats, idx_t, zrow)

    W2p = jnp.zeros((D, D), jnp.float32).at[:, :H_OUT].set(W2)
    b2p = jnp.zeros((1, D), jnp.float32).at[0, :H_OUT].set(b2)
    out = _tc_bn_mlp(
        partials,
        gamma.reshape(1, D),
        beta.reshape(1, D),
        W1,
        b1.reshape(1, D),
        W2p,
        b2p,
    )
    return out[:, :H_OUT]


# speedup vs baseline: 1.0209x; 1.0209x over previous
import functools

import jax
import jax.numpy as jnp
from jax import lax
from jax.experimental import pallas as pl
from jax.experimental.pallas import tpu as pltpu
from jax.experimental.pallas import tpu_sc as plsc

N = 100000
D = 128
G = 4096
H_OUT = 12

NC = 2
NS = 16
NW = NC * NS
CHUNK = 128
NFULL = N // CHUNK
TAIL = N - NFULL * CHUNK
JMAX = 25
ROWS_PER_SID = G // NS


def _sc_segment_sum(node_feats, idx_t, zrow):
    mesh = plsc.VectorSubcoreMesh(
        core_axis_name="c", subcore_axis_name="s", num_cores=NC, num_subcores=NS
    )

    @functools.partial(
        pl.kernel,
        out_type=jax.ShapeDtypeStruct((NC, G, D), jnp.float32),
        mesh=mesh,
        scratch_types=[
            pltpu.VMEM((JMAX, CHUNK), jnp.int32),
            pltpu.VMEM((4, CHUNK, D), jnp.float32),
            pltpu.VMEM((CHUNK, D), jnp.float32),
            pltpu.VMEM_SHARED((G, D), jnp.float32),
            pltpu.SemaphoreType.DMA((4,)),
            pltpu.SemaphoreType.DMA,
        ],
    )
    def seg_sum(node_hbm, idx_hbm, zrow_hbm, out_hbm, ids_buf, rbuf, zbuf, acc,
                sem, sem_s):
        cid = lax.axis_index("c")
        sid = lax.axis_index("s")
        wid = cid * NS + sid

        pltpu.sync_copy(idx_hbm.at[wid], ids_buf)
        pltpu.sync_copy(zrow_hbm, zbuf)

        base = sid * ROWS_PER_SID
        pltpu.sync_copy(zbuf, acc.at[pl.ds(base, CHUNK)])
        pltpu.sync_copy(zbuf, acc.at[pl.ds(base + CHUNK, CHUNK)])
        plsc.subcore_barrier()

        nfull = jnp.where(wid <= 12, JMAX, JMAX - 1)

        for p in range(2):
            pltpu.async_copy(
                node_hbm.at[pl.ds((wid + NW * p) * CHUNK, CHUNK)],
                rbuf.at[p], sem.at[p],
            )

        def body(j, carry):
            b = lax.rem(j, 4)

            @pl.when(j >= 2)
            def _():
                pltpu.make_async_copy(
                    rbuf.at[0], acc.at[ids_buf.at[0]], sem_s
                ).wait()

            @pl.when(j + 2 < nfull)
            def _():
                c2 = wid + NW * (j + 2)
                b2 = lax.rem(j + 2, 4)
                pltpu.async_copy(
                    node_hbm.at[pl.ds(c2 * CHUNK, CHUNK)], rbuf.at[b2], sem.at[b2]
                )

            pltpu.make_async_copy(
                node_hbm.at[pl.ds(0, CHUNK)], rbuf.at[b], sem.at[b]
            ).wait()
            pltpu.async_copy(rbuf.at[b], acc.at[ids_buf.at[j]], sem_s, priority=1, add=True)
            return carry

        lax.fori_loop(0, nfull, body, 0)

        for _ in range(2):
            pltpu.make_async_copy(rbuf.at[0], acc.at[ids_buf.at[0]], sem_s).wait()

        @pl.when(wid == 13)
        def _():
            pltpu.sync_copy(
                node_hbm.at[pl.ds(NFULL * CHUNK, TAIL)], zbuf.at[pl.ds(0, TAIL)]
            )
            pltpu.sync_copy(zbuf, acc.at[ids_buf.at[JMAX - 1]], add=True)

        plsc.subcore_barrier()

        pltpu.sync_copy(
            acc.at[pl.ds(base, ROWS_PER_SID)],
            out_hbm.at[cid, pl.ds(base, ROWS_PER_SID)],
        )

    return seg_sum(node_feats, idx_t, zrow)


def _tc_body(p_ref, g_ref, b_ref, w1_ref, b1_ref, w2_ref, b2_ref, o_ref):
    x = p_ref[0] + p_ref[1]
    mean = jnp.mean(x, axis=0, keepdims=True)
    xc = x - mean
    var = jnp.mean(xc * xc, axis=0, keepdims=True)
    gn = xc * lax.rsqrt(var + 1e-5) * g_ref[...] + b_ref[...]
    h = jnp.dot(gn, w1_ref[...], preferred_element_type=jnp.float32) + b1_ref[...]
    h = jnp.maximum(h, 0.0)
    o_ref[...] = jnp.dot(h, w2_ref[...], preferred_element_type=jnp.float32) + b2_ref[...]


def _tc_bn_mlp(partials, gamma, beta, W1, b1, W2p, b2p):
    return pl.pallas_call(
        _tc_body,
        out_shape=jax.ShapeDtypeStruct((G, D), jnp.float32),
    )(partials, gamma, beta, W1, b1, W2p, b2p)


def kernel(node_feats, segment_ids, gamma, beta, W1, b1, W2, b2):
    ids32 = segment_ids.astype(jnp.int32)
    ids_pad = jnp.zeros((NW * JMAX * CHUNK,), jnp.int32).at[:N].set(ids32)
    idx_t = ids_pad.reshape(JMAX, NW, CHUNK).transpose(1, 0, 2)
    zrow = jnp.zeros((CHUNK, D), jnp.float32)

    partials = _sc_segment_sum(node_feats, idx_t, zrow)

    W2p = jnp.zeros((D, D), jnp.float32).at[:, :H_OUT].set(W2)
    b2p = jnp.zeros((1, D), jnp.float32).at[0, :H_OUT].set(b2)
    out = _tc_bn_mlp(
        partials,
        gamma.reshape(1, D),
        beta.reshape(1, D),
        W1,
        b1.reshape(1, D),
        W2p,
        b2p,
    )
    return out[:, :H_OUT]
